# baseline (speedup 1.0000x reference)
import functools

import jax
import jax.numpy as jnp
from jax import lax
from jax.experimental import pallas as pl
from jax.experimental.pallas import tpu as pltpu
from jax.experimental.pallas import tpu_sc as plsc

GRID_NUNQ = 384
HIDDEN = 128
HIST = 200
TB = 128

_SC_INFO = plsc.get_sparse_core_info()
_NC = _SC_INFO.num_cores
_NS = _SC_INFO.num_subcores
_NW = _NC * _NS


def _sc_gather(table_hbm, idx_hbm, out_hbm, idx_v, rows_v, sem):
    b_per_w = idx_v.shape[0]
    wid = lax.axis_index("s") * _NC + lax.axis_index("c")
    base = wid * b_per_w
    pltpu.sync_copy(idx_hbm.at[pl.ds(base, b_per_w)], idx_v)
    pltpu.async_copy(table_hbm.at[idx_v], rows_v, sem).wait()
    pltpu.sync_copy(rows_v, out_hbm.at[pl.ds(base, b_per_w)])


def _gather_rows(embedding, g_id):
    batch = g_id.shape[0]
    b_per_w = batch // _NW
    mesh = plsc.VectorSubcoreMesh(core_axis_name="c", subcore_axis_name="s")
    return pl.kernel(
        _sc_gather,
        mesh=mesh,
        out_type=jax.ShapeDtypeStruct((batch, HIDDEN), jnp.float32),
        scratch_types=[
            pltpu.VMEM((b_per_w,), jnp.int32),
            pltpu.VMEM((b_per_w, HIDDEN), jnp.float32),
            pltpu.SemaphoreType.DMA,
        ],
    )(embedding, g_id)


def _tc_add(x_ref, eg_ref, o_ref):
    o_ref[...] = x_ref[...] + eg_ref[...][:, None, :]


@jax.jit
def kernel(x, g_id, embedding):
    batch = x.shape[0]
    num_blocks = batch // TB
    emb_g = _gather_rows(embedding, g_id.astype(jnp.int32))
    return pl.pallas_call(
        _tc_add,
        grid=(num_blocks,),
        in_specs=[
            pl.BlockSpec((TB, HIST, HIDDEN), lambda i: (i, 0, 0)),
            pl.BlockSpec((TB, HIDDEN), lambda i: (i, 0)),
        ],
        out_specs=pl.BlockSpec((TB, HIST, HIDDEN), lambda i: (i, 0, 0)),
        out_shape=jax.ShapeDtypeStruct((batch, HIST, HIDDEN), jnp.float32),
        compiler_params=pltpu.CompilerParams(
            dimension_semantics=("arbitrary",),
        ),
    )(x, emb_g)
